# baseline (speedup 1.0000x reference)
import functools

import jax
import jax.numpy as jnp
from jax import lax
from jax.experimental import pallas as pl
from jax.experimental.pallas import tpu as pltpu
from jax.experimental.pallas import tpu_sc as plsc

_VOCAB = 1000
_DIM = 64
_B = 4096
_WIN = 200
_VP = 1024
_WP = 256
_NC, _NS = 2, 16
_NW = _NC * _NS
_RPW = _B // _NW
_CH = 32
_NCHUNK = _RPW // _CH
_NJ = 14
_OW = 32
_TR = _VOCAB // 8


def _g_body(e_ref, w_ref, bias_ref, seen_ref, g_ref, s2_ref):
    a = lax.dot_general(e_ref[...], w_ref[...], (((1,), (1,)), ((), ())),
                        preferred_element_type=jnp.float32,
                        precision=lax.Precision.HIGHEST)
    g = lax.dot_general(a, e_ref[...], (((1,), (1,)), ((), ())),
                        preferred_element_type=jnp.float32,
                        precision=lax.Precision.HIGHEST)
    b = bias_ref[...]
    gp = jnp.concatenate(
        [g + b, jnp.broadcast_to(b, (_VOCAB, _VP - _VOCAB))], axis=1)
    g_ref[...] = gp.reshape(_TR, 8, 8, 128).transpose(0, 2, 1, 3)
    s = seen_ref[...]
    s2_ref[0, :, :] = s[:, :128]
    s2_ref[1, :, : _WIN - 128] = s[:, 128:]
    s2_ref[1, :, _WIN - 128 :] = jnp.full((_B, 128 - (_WIN - 128)), _VOCAB,
                                          dtype=jnp.int32)


_g_call = pl.pallas_call(
    _g_body,
    in_specs=[
        pl.BlockSpec((_VOCAB, _DIM), lambda: (0, 0)),
        pl.BlockSpec((_DIM, _DIM), lambda: (0, 0)),
        pl.BlockSpec((_VOCAB, 1), lambda: (0, 0)),
        pl.BlockSpec((_B, _WIN), lambda: (0, 0)),
    ],
    out_specs=[
        pl.BlockSpec((_TR, 8, 8, 128), lambda: (0, 0, 0, 0)),
        pl.BlockSpec((2, _B, 128), lambda: (0, 0, 0)),
    ],
    out_shape=[
        jax.ShapeDtypeStruct((_TR, 8, 8, 128), jnp.float32),
        jax.ShapeDtypeStruct((2, _B, 128), jnp.int32),
    ],
)


_sc_mesh = plsc.VectorSubcoreMesh(
    core_axis_name="c", subcore_axis_name="s", num_cores=_NC, num_subcores=_NS
)


@functools.partial(
    pl.kernel,
    out_type=jax.ShapeDtypeStruct((_B * _OW,), jnp.float32),
    mesh=_sc_mesh,
    scratch_types=[
        pltpu.VMEM((_RPW,), jnp.int32),
        [pltpu.VMEM((8 * _CH,), jnp.int32) for _ in range(2)],
        [pltpu.VMEM((8 * _CH, 128), jnp.float32) for _ in range(2)],
        pltpu.VMEM((2 * _RPW, 128), jnp.int32),
        pltpu.VMEM((_RPW * _OW,), jnp.float32),
        [pltpu.SemaphoreType.DMA for _ in range(2)],
        pltpu.SemaphoreType.DMA,
        pltpu.SemaphoreType.DMA,
    ],
    compiler_params=pltpu.CompilerParams(
        use_tc_tiling_on_sc=False, needs_layout_passes=False
    ),
)
def _sc_gather(g_hbm, seen_hbm, dec_hbm, out_hbm,
               dec_v, idx_v, rows_v, seen_v, out_v, sem_g, sem_s, sem_o):
    wid = lax.axis_index("s") * _NC + lax.axis_index("c")
    base0 = wid * _RPW
    lane = lax.iota(jnp.int32, 16)
    lane_hi = lane >> 3
    lane_j3 = (lane & 7) << 3

    pltpu.sync_copy(dec_hbm.at[pl.ds(base0, _RPW)], dec_v)
    h_s0 = pltpu.async_copy(seen_hbm.at[pl.ds(base0, _RPW)],
                            seen_v.at[pl.ds(0, _RPW)], sem_s)
    h_s1 = pltpu.async_copy(seen_hbm.at[pl.ds(_B + base0, _RPW)],
                            seen_v.at[pl.ds(_RPW, _RPW)], sem_s)

    def fetch(ci, buf):
        idx = idx_v[buf]
        for m in range(_CH // 2):
            d = plsc.load_gather(dec_v, [lane_hi + (ci * _CH + 2 * m)])
            idx[pl.ds(16 * m, 16)] = ((d >> 3) << 6) + lane_j3 + (d & 7)
        h0 = pltpu.async_copy(g_hbm.at[idx.at[pl.ds(0, 128)]],
                              rows_v[buf].at[pl.ds(0, 128)], sem_g[buf])
        h1 = pltpu.async_copy(g_hbm.at[idx.at[pl.ds(128, 128)]],
                              rows_v[buf].at[pl.ds(128, 128)], sem_g[buf])
        return (h0, h1)

    h_g = [None, None]
    h_g[0] = fetch(0, 0)
    h_s0.wait()
    h_s1.wait()
    for ci in range(_NCHUNK):
        cur = ci % 2
        nxt = (ci + 1) % 2
        if ci + 1 < _NCHUNK:
            h_g[nxt] = fetch(ci + 1, nxt)
        h_g[cur][0].wait()
        h_g[cur][1].wait()
        rows = rows_v[cur]
        rbase = ci * _CH

        @plsc.parallel_loop(0, _CH, unroll=8)
        def row_body(r):
            r8 = jnp.full((16,), 8 * r, dtype=jnp.int32)
            acc = None
            for j in range(_NJ):
                cols = seen_v[(j >> 3) * _RPW + rbase + r, pl.ds((j & 7) * 16, 16)]
                t = plsc.load_gather(rows, [r8 + (cols >> 7), cols & 127])
                if j < _NJ - 1:
                    acc = t if acc is None else acc * t
                else:
                    out_v[pl.ds((rbase + r) * _OW, 16)] = acc
                    out_v[pl.ds((rbase + r) * _OW + 16, 16)] = t

    pltpu.async_copy(out_v, out_hbm.at[pl.ds(base0 * _OW, _RPW * _OW)],
                     sem_o).wait()


def _reduce_body(t_ref, o_ref):
    t = t_ref[...]
    logs = jnp.log(t)
    l_iota = lax.broadcasted_iota(jnp.int32, (128, 8), 0)
    q_iota = lax.broadcasted_iota(jnp.int32, (128, 8), 1)
    seg = l_iota // 32
    off = l_iota % 32
    m_sum = jnp.where((q_iota == seg) & (off < 16), 1.0, 0.0)
    m_bias = jnp.where((q_iota - 4 == seg) & (off == 16), 1.0, 0.0)
    s = jnp.dot(logs, m_sum + m_bias,
                preferred_element_type=jnp.float32,
                precision=lax.Precision.HIGHEST)
    o_ref[...] = jnp.exp(s[:, 0:4] - 207.0 * s[:, 4:8])


_reduce_call = pl.pallas_call(
    _reduce_body,
    in_specs=[pl.BlockSpec((_B * _OW // 128, 128), lambda: (0, 0))],
    out_specs=pl.BlockSpec((_B * _OW // 128, 4), lambda: (0, 0)),
    out_shape=jax.ShapeDtypeStruct((_B * _OW // 128, 4), jnp.float32),
)


@jax.jit
def kernel(user, seen, seen_users, decision, emb_table, emb_bias_table, blinear_w):
    del user, seen_users
    g, seen2 = _g_call(emb_table, blinear_w, emb_bias_table,
                       seen.astype(jnp.int32))
    pb = _sc_gather(g.reshape(8 * _VOCAB, 128), seen2.reshape(2 * _B, 128),
                    decision.astype(jnp.int32))
    o2 = _reduce_call(pb.reshape(_B * _OW // 128, 128))
    return o2.reshape(_B)


# speedup vs baseline: 1.0493x; 1.0493x over previous
import functools

import jax
import jax.numpy as jnp
from jax import lax
from jax.experimental import pallas as pl
from jax.experimental.pallas import tpu as pltpu
from jax.experimental.pallas import tpu_sc as plsc

_VOCAB = 1000
_DIM = 64
_B = 4096
_WIN = 200
_VP = 1024
_WP = 256
_NC, _NS = 2, 16
_NW = _NC * _NS
_RPW = _B // _NW
_CH = 32
_NCHUNK = _RPW // _CH
_NJ = 14
_OW = 32
_TR = _VOCAB // 8


def _g_body(et_ref, w_ref, bias_ref, g_ref):
    et = et_ref[...]
    a = lax.dot_general(et, w_ref[...], (((0,), (1,)), ((), ())),
                        preferred_element_type=jnp.float32,
                        precision=lax.Precision.HIGHEST)
    g = lax.dot_general(a, et, (((1,), (0,)), ((), ())),
                        preferred_element_type=jnp.float32,
                        precision=lax.Precision.HIGHEST)
    b = bias_ref[...].reshape(1, _VOCAB).T
    gp = jnp.concatenate(
        [g + b, jnp.broadcast_to(b, (_VOCAB, _VP - _VOCAB))], axis=1)
    g_ref[...] = gp.reshape(_TR, 8, 8, 128).transpose(0, 2, 1, 3)


_g_call = pl.pallas_call(
    _g_body,
    in_specs=[
        pl.BlockSpec((_DIM, _VOCAB), lambda: (0, 0)),
        pl.BlockSpec((_DIM, _DIM), lambda: (0, 0)),
        pl.BlockSpec((1, _VOCAB), lambda: (0, 0)),
    ],
    out_specs=pl.BlockSpec((_TR, 8, 8, 128), lambda: (0, 0, 0, 0)),
    out_shape=jax.ShapeDtypeStruct((_TR, 8, 8, 128), jnp.float32),
)


_sc_mesh = plsc.VectorSubcoreMesh(
    core_axis_name="c", subcore_axis_name="s", num_cores=_NC, num_subcores=_NS
)


@functools.partial(
    pl.kernel,
    out_type=jax.ShapeDtypeStruct((_B * _OW,), jnp.float32),
    mesh=_sc_mesh,
    scratch_types=[
        pltpu.VMEM((_RPW,), jnp.int32),
        [pltpu.VMEM((8 * _CH,), jnp.int32) for _ in range(2)],
        [pltpu.VMEM((8 * _CH, 128), jnp.float32) for _ in range(2)],
        pltpu.VMEM((2 * _RPW, 128), jnp.int32),
        pltpu.VMEM((_RPW * _OW,), jnp.float32),
        [pltpu.SemaphoreType.DMA for _ in range(2)],
        pltpu.SemaphoreType.DMA,
        pltpu.SemaphoreType.DMA,
    ],
    compiler_params=pltpu.CompilerParams(
        use_tc_tiling_on_sc=False, needs_layout_passes=False
    ),
)
def _sc_gather(g_hbm, seen_hbm, dec_hbm, out_hbm,
               dec_v, idx_v, rows_v, seen_v, out_v, sem_g, sem_s, sem_o):
    wid = lax.axis_index("s") * _NC + lax.axis_index("c")
    base0 = wid * _RPW
    lane = lax.iota(jnp.int32, 16)
    lane_hi = lane >> 3
    lane_j3 = (lane & 7) << 3

    pltpu.sync_copy(dec_hbm.at[pl.ds(base0, _RPW)], dec_v)
    h_s = pltpu.async_copy(seen_hbm.at[pl.ds(2 * base0, 2 * _RPW)],
                           seen_v, sem_s)

    def fetch(ci, buf):
        idx = idx_v[buf]
        for m in range(_CH // 2):
            d = plsc.load_gather(dec_v, [lane_hi + (ci * _CH + 2 * m)])
            idx[pl.ds(16 * m, 16)] = ((d >> 3) << 6) + lane_j3 + (d & 7)
        h0 = pltpu.async_copy(g_hbm.at[idx.at[pl.ds(0, 128)]],
                              rows_v[buf].at[pl.ds(0, 128)], sem_g[buf])
        h1 = pltpu.async_copy(g_hbm.at[idx.at[pl.ds(128, 128)]],
                              rows_v[buf].at[pl.ds(128, 128)], sem_g[buf])
        return (h0, h1)

    h_g = [None, None]
    h_g[0] = fetch(0, 0)
    h_s.wait()
    for ci in range(_NCHUNK):
        cur = ci % 2
        nxt = (ci + 1) % 2
        if ci + 1 < _NCHUNK:
            h_g[nxt] = fetch(ci + 1, nxt)
        h_g[cur][0].wait()
        h_g[cur][1].wait()
        rows = rows_v[cur]
        rbase = ci * _CH

        @plsc.parallel_loop(0, _CH, unroll=8)
        def row_body(r):
            r8 = jnp.full((16,), 8 * r, dtype=jnp.int32)
            acc = None
            for j in range(_NJ):
                cols = seen_v[2 * (rbase + r) + (j >> 3), pl.ds((j & 7) * 16, 16)]
                t = plsc.load_gather(rows, [r8 + (cols >> 7), cols & 127])
                if j < _NJ - 1:
                    acc = t if acc is None else acc * t
                else:
                    out_v[pl.ds((rbase + r) * _OW, 16)] = acc
                    out_v[pl.ds((rbase + r) * _OW + 16, 16)] = t

    pltpu.async_copy(out_v, out_hbm.at[pl.ds(base0 * _OW, _RPW * _OW)],
                     sem_o).wait()


def _reduce_body(t_ref, o_ref):
    t = t_ref[...]
    logs = jnp.log(t)
    l_iota = lax.broadcasted_iota(jnp.int32, (128, 8), 0)
    q_iota = lax.broadcasted_iota(jnp.int32, (128, 8), 1)
    seg = l_iota // 32
    off = l_iota % 32
    m_sum = jnp.where((q_iota == seg) & (off < 16), 1.0, 0.0)
    m_bias = jnp.where((q_iota - 4 == seg) & (off == 16), 1.0, 0.0)
    s = jnp.dot(logs, m_sum + m_bias,
                preferred_element_type=jnp.float32,
                precision=lax.Precision.HIGHEST)
    o_ref[...] = jnp.exp(s[:, 0:4] - 207.0 * s[:, 4:8])


_reduce_call = pl.pallas_call(
    _reduce_body,
    in_specs=[pl.BlockSpec((_B * _OW // 128, 128), lambda: (0, 0))],
    out_specs=pl.BlockSpec((_B * _OW // 128, 4), lambda: (0, 0)),
    out_shape=jax.ShapeDtypeStruct((_B * _OW // 128, 4), jnp.float32),
)


@jax.jit
def kernel(user, seen, seen_users, decision, emb_table, emb_bias_table, blinear_w):
    del user, seen_users
    g = _g_call(emb_table.T, blinear_w, emb_bias_table.T)
    seen_pad = jnp.pad(seen.astype(jnp.int32), ((0, 0), (0, _WP - _WIN)),
                       constant_values=_VOCAB).reshape(2 * _B, 128)
    pb = _sc_gather(g.reshape(8 * _VOCAB, 128), seen_pad,
                    decision.astype(jnp.int32))
    o2 = _reduce_call(pb.reshape(_B * _OW // 128, 128))
    return o2.reshape(_B)


# speedup vs baseline: 1.0903x; 1.0390x over previous
import functools

import jax
import jax.numpy as jnp
from jax import lax
from jax.experimental import pallas as pl
from jax.experimental.pallas import tpu as pltpu
from jax.experimental.pallas import tpu_sc as plsc

_VOCAB = 1000
_DIM = 64
_B = 4096
_WIN = 200
_VP = 1024
_WP = 256
_NC, _NS = 2, 16
_NW = _NC * _NS
_RPW = _B // _NW
_CH = 32
_NCHUNK = _RPW // _CH
_NJ = 14
_OW = 32
_TR = _VOCAB // 8


def _g_body(et_ref, w_ref, bias_ref, g_ref):
    et = et_ref[...]
    a2 = lax.dot_general(w_ref[...], et, (((1,), (0,)), ((), ())),
                         preferred_element_type=jnp.float32,
                         precision=lax.Precision.HIGHEST)
    g = lax.dot_general(a2, et, (((0,), (0,)), ((), ())),
                        preferred_element_type=jnp.float32,
                        precision=lax.Precision.HIGHEST)
    b = bias_ref[...].reshape(1, _VOCAB).T
    gp = jnp.concatenate(
        [g + b, jnp.broadcast_to(b, (_VOCAB, _VP - _VOCAB))], axis=1)
    g_ref[...] = gp.reshape(_TR, 8, 8, 128).transpose(0, 2, 1, 3)


_g_call = pl.pallas_call(
    _g_body,
    in_specs=[
        pl.BlockSpec((_DIM, _VOCAB), lambda: (0, 0)),
        pl.BlockSpec((_DIM, _DIM), lambda: (0, 0)),
        pl.BlockSpec((1, _VOCAB), lambda: (0, 0)),
    ],
    out_specs=pl.BlockSpec((_TR, 8, 8, 128), lambda: (0, 0, 0, 0)),
    out_shape=jax.ShapeDtypeStruct((_TR, 8, 8, 128), jnp.float32),
)


_sc_mesh = plsc.VectorSubcoreMesh(
    core_axis_name="c", subcore_axis_name="s", num_cores=_NC, num_subcores=_NS
)


@functools.partial(
    pl.kernel,
    out_type=jax.ShapeDtypeStruct((_B * _OW,), jnp.float32),
    mesh=_sc_mesh,
    scratch_types=[
        pltpu.VMEM((_RPW,), jnp.int32),
        [pltpu.VMEM((8 * _CH,), jnp.int32) for _ in range(2)],
        [pltpu.VMEM((8 * _CH, 128), jnp.float32) for _ in range(2)],
        pltpu.VMEM((2 * _RPW, 128), jnp.int32),
        pltpu.VMEM((_RPW * _OW,), jnp.float32),
        [pltpu.SemaphoreType.DMA for _ in range(2)],
        pltpu.SemaphoreType.DMA,
        pltpu.SemaphoreType.DMA,
    ],
    compiler_params=pltpu.CompilerParams(
        use_tc_tiling_on_sc=False, needs_layout_passes=False
    ),
)
def _sc_gather(g_hbm, seen_hbm, dec_hbm, out_hbm,
               dec_v, idx_v, rows_v, seen_v, out_v, sem_g, sem_s, sem_o):
    wid = lax.axis_index("s") * _NC + lax.axis_index("c")
    base0 = wid * _RPW
    lane = lax.iota(jnp.int32, 16)
    lane_hi = lane >> 3
    lane_j3 = (lane & 7) << 3

    pltpu.sync_copy(dec_hbm.at[pl.ds(base0, _RPW)], dec_v)
    h_s = pltpu.async_copy(seen_hbm.at[pl.ds(2 * base0, 2 * _RPW)],
                           seen_v, sem_s)

    def fetch(ci, buf):
        idx = idx_v[buf]
        for m in range(_CH // 2):
            d = plsc.load_gather(dec_v, [lane_hi + (ci * _CH + 2 * m)])
            idx[pl.ds(16 * m, 16)] = ((d >> 3) << 6) + lane_j3 + (d & 7)
        h0 = pltpu.async_copy(g_hbm.at[idx.at[pl.ds(0, 128)]],
                              rows_v[buf].at[pl.ds(0, 128)], sem_g[buf])
        h1 = pltpu.async_copy(g_hbm.at[idx.at[pl.ds(128, 128)]],
                              rows_v[buf].at[pl.ds(128, 128)], sem_g[buf])
        return (h0, h1)

    h_g = [None, None]
    h_g[0] = fetch(0, 0)
    h_s.wait()
    for ci in range(_NCHUNK):
        cur = ci % 2
        nxt = (ci + 1) % 2
        if ci + 1 < _NCHUNK:
            h_g[nxt] = fetch(ci + 1, nxt)
        h_g[cur][0].wait()
        h_g[cur][1].wait()
        rows = rows_v[cur]
        rbase = ci * _CH

        @plsc.parallel_loop(0, _CH, unroll=8)
        def row_body(r):
            r8 = jnp.full((16,), 8 * r, dtype=jnp.int32)
            acc = None
            for j in range(_NJ):
                cols = seen_v[2 * (rbase + r) + (j >> 3), pl.ds((j & 7) * 16, 16)]
                t = plsc.load_gather(rows, [r8 + (cols >> 7), cols & 127])
                if j < _NJ - 1:
                    acc = t if acc is None else acc * t
                else:
                    out_v[pl.ds((rbase + r) * _OW, 16)] = acc
                    out_v[pl.ds((rbase + r) * _OW + 16, 16)] = t

    pltpu.async_copy(out_v, out_hbm.at[pl.ds(base0 * _OW, _RPW * _OW)],
                     sem_o).wait()


def _reduce_body(t_ref, o_ref):
    t = t_ref[...]
    logs = jnp.log(t)
    l_iota = lax.broadcasted_iota(jnp.int32, (128, 8), 0)
    q_iota = lax.broadcasted_iota(jnp.int32, (128, 8), 1)
    seg = l_iota // 32
    off = l_iota % 32
    m_sum = jnp.where((q_iota == seg) & (off < 16), 1.0, 0.0)
    m_bias = jnp.where((q_iota - 4 == seg) & (off == 16), 1.0, 0.0)
    s = jnp.dot(logs, m_sum + m_bias,
                preferred_element_type=jnp.float32,
                precision=lax.Precision.HIGHEST)
    o_ref[...] = jnp.exp(s[:, 0:4] - 207.0 * s[:, 4:8])


_reduce_call = pl.pallas_call(
    _reduce_body,
    in_specs=[pl.BlockSpec((_B * _OW // 128, 128), lambda: (0, 0))],
    out_specs=pl.BlockSpec((_B * _OW // 128, 4), lambda: (0, 0)),
    out_shape=jax.ShapeDtypeStruct((_B * _OW // 128, 4), jnp.float32),
)


@jax.jit
def kernel(user, seen, seen_users, decision, emb_table, emb_bias_table, blinear_w):
    del user, seen_users
    g = _g_call(emb_table.T, blinear_w, emb_bias_table.T)
    seen_pad = jnp.pad(seen.astype(jnp.int32), ((0, 0), (0, _WP - _WIN)),
                       constant_values=_VOCAB).reshape(2 * _B, 128)
    pb = _sc_gather(g.reshape(8 * _VOCAB, 128), seen_pad,
                    decision.astype(jnp.int32))
    o2 = _reduce_call(pb.reshape(_B * _OW // 128, 128))
    return o2.reshape(_B)


# speedup vs baseline: 1.1079x; 1.0161x over previous
import functools

import jax
import jax.numpy as jnp
from jax import lax
from jax.experimental import pallas as pl
from jax.experimental.pallas import tpu as pltpu
from jax.experimental.pallas import tpu_sc as plsc

_VOCAB = 1000
_DIM = 64
_B = 4096
_WIN = 200
_VP = 1024
_WP = 256
_NC, _NS = 2, 16
_NW = _NC * _NS
_RPW = _B // _NW
_CH = 32
_NCHUNK = _RPW // _CH
_NJ = 14
_OW = 32
_TR = _VOCAB // 8


def _g_body(et_ref, w_ref, bias_ref, g_ref):
    et = et_ref[...]
    a2 = lax.dot_general(w_ref[...], et, (((1,), (0,)), ((), ())),
                         preferred_element_type=jnp.float32,
                         precision=lax.Precision.HIGHEST)
    g = lax.dot_general(a2, et, (((0,), (0,)), ((), ())),
                        preferred_element_type=jnp.float32,
                        precision=lax.Precision.HIGHEST)
    b = bias_ref[...].reshape(1, _VOCAB).T
    gp = jnp.concatenate(
        [g + b, jnp.broadcast_to(b, (_VOCAB, _VP - _VOCAB))], axis=1)
    g_ref[...] = gp.reshape(_TR, 8, 8, 128).transpose(0, 2, 1, 3)


_g_call = pl.pallas_call(
    _g_body,
    in_specs=[
        pl.BlockSpec((_DIM, _VOCAB), lambda: (0, 0)),
        pl.BlockSpec((_DIM, _DIM), lambda: (0, 0)),
        pl.BlockSpec((1, _VOCAB), lambda: (0, 0)),
    ],
    out_specs=pl.BlockSpec((_TR, 8, 8, 128), lambda: (0, 0, 0, 0)),
    out_shape=jax.ShapeDtypeStruct((_TR, 8, 8, 128), jnp.float32),
)


_sc_mesh = plsc.VectorSubcoreMesh(
    core_axis_name="c", subcore_axis_name="s", num_cores=_NC, num_subcores=_NS
)


@functools.partial(
    pl.kernel,
    out_type=jax.ShapeDtypeStruct((_B * _OW,), jnp.float32),
    mesh=_sc_mesh,
    scratch_types=[
        pltpu.VMEM((_RPW,), jnp.int32),
        [pltpu.VMEM((8 * _CH,), jnp.int32) for _ in range(2)],
        [pltpu.VMEM((8 * _CH, 128), jnp.float32) for _ in range(2)],
        pltpu.VMEM((2 * _RPW, 128), jnp.int32),
        pltpu.VMEM((_RPW * _OW,), jnp.float32),
        [pltpu.SemaphoreType.DMA for _ in range(2)],
        pltpu.SemaphoreType.DMA,
        pltpu.SemaphoreType.DMA,
    ],
    compiler_params=pltpu.CompilerParams(
        use_tc_tiling_on_sc=False, needs_layout_passes=False
    ),
)
def _sc_gather(g_hbm, seen_hbm, dec_hbm, out_hbm,
               dec_v, idx_v, rows_v, seen_v, out_v, sem_g, sem_s, sem_o):
    wid = lax.axis_index("s") * _NC + lax.axis_index("c")
    base0 = wid * _RPW
    lane = lax.iota(jnp.int32, 16)
    lane_hi = lane >> 3
    lane_j3 = (lane & 7) << 3

    pltpu.sync_copy(dec_hbm.at[pl.ds(base0, _RPW)], dec_v)
    h_s = pltpu.async_copy(seen_hbm.at[pl.ds(2 * base0, 2 * _RPW)],
                           seen_v, sem_s)

    def fetch(ci, buf):
        idx = idx_v[buf]
        for m in range(_CH // 2):
            d = plsc.load_gather(dec_v, [lane_hi + (ci * _CH + 2 * m)])
            idx[pl.ds(16 * m, 16)] = ((d >> 3) << 6) + lane_j3 + (d & 7)
        h0 = pltpu.async_copy(g_hbm.at[idx.at[pl.ds(0, 128)]],
                              rows_v[buf].at[pl.ds(0, 128)], sem_g[buf])
        h1 = pltpu.async_copy(g_hbm.at[idx.at[pl.ds(128, 128)]],
                              rows_v[buf].at[pl.ds(128, 128)], sem_g[buf])
        return (h0, h1)

    h_g = [None, None]
    h_g[0] = fetch(0, 0)
    h_s.wait()
    for ci in range(_NCHUNK):
        cur = ci % 2
        nxt = (ci + 1) % 2
        if ci + 1 < _NCHUNK:
            h_g[nxt] = fetch(ci + 1, nxt)
        h_g[cur][0].wait()
        h_g[cur][1].wait()
        rows = rows_v[cur]
        rbase = ci * _CH

        @plsc.parallel_loop(0, _CH, unroll=4)
        def row_body(r):
            r8 = jnp.full((16,), 8 * r, dtype=jnp.int32)
            acc = None
            for j in range(_NJ):
                cols = seen_v[2 * (rbase + r) + (j >> 3), pl.ds((j & 7) * 16, 16)]
                t = plsc.load_gather(rows, [r8 + (cols >> 7), cols & 127])
                if j < _NJ - 1:
                    acc = t if acc is None else acc * t
                else:
                    out_v[pl.ds((rbase + r) * _OW, 16)] = acc
                    out_v[pl.ds((rbase + r) * _OW + 16, 16)] = t

    pltpu.async_copy(out_v, out_hbm.at[pl.ds(base0 * _OW, _RPW * _OW)],
                     sem_o).wait()


def _reduce_body(t_ref, o_ref):
    t = t_ref[...]
    logs = jnp.log(t)
    l_iota = lax.broadcasted_iota(jnp.int32, (128, 8), 0)
    q_iota = lax.broadcasted_iota(jnp.int32, (128, 8), 1)
    seg = l_iota // 32
    off = l_iota % 32
    m_sum = jnp.where((q_iota == seg) & (off < 16), 1.0, 0.0)
    m_bias = jnp.where((q_iota - 4 == seg) & (off == 16), 1.0, 0.0)
    s = jnp.dot(logs, m_sum + m_bias,
                preferred_element_type=jnp.float32,
                precision=lax.Precision.HIGHEST)
    o_ref[...] = jnp.exp(s[:, 0:4] - 207.0 * s[:, 4:8])


_reduce_call = pl.pallas_call(
    _reduce_body,
    in_specs=[pl.BlockSpec((_B * _OW // 128, 128), lambda: (0, 0))],
    out_specs=pl.BlockSpec((_B * _OW // 128, 4), lambda: (0, 0)),
    out_shape=jax.ShapeDtypeStruct((_B * _OW // 128, 4), jnp.float32),
)


@jax.jit
def kernel(user, seen, seen_users, decision, emb_table, emb_bias_table, blinear_w):
    del user, seen_users
    g = _g_call(emb_table.T, blinear_w, emb_bias_table.T)
    seen_pad = jnp.pad(seen.astype(jnp.int32), ((0, 0), (0, _WP - _WIN)),
                       constant_values=_VOCAB).reshape(2 * _B, 128)
    pb = _sc_gather(g.reshape(8 * _VOCAB, 128), seen_pad,
                    decision.astype(jnp.int32))
    o2 = _reduce_call(pb.reshape(_B * _OW // 128, 128))
    return o2.reshape(_B)
